# Initial kernel scaffold; baseline (speedup 1.0000x reference)
import jax
import jax.numpy as jnp
from jax.experimental import pallas as pl


def kernel(zij_label, rij, nuww, sigmas, centres):
    raise NotImplementedError("write your pallas kernel here")


# speedup vs baseline: 2.8753x; 2.8753x over previous
import functools

import jax
import jax.numpy as jnp
from jax import lax
from jax.experimental import pallas as pl
from jax.experimental.pallas import tpu as pltpu
from jax.experimental.pallas import tpu_sc as plsc

_NC = 2
_NS = 16
_NW = _NC * _NS
_LANES = 16


def _rconv_body(chunk, n_out_vecs,
                zij_hbm, rij_hbm, params_hbm, centres_hbm,
                out_hbm,
                params_v, centres_v, zij_v, rij_v, out_v):
    wid = lax.axis_index("s") * _NC + lax.axis_index("c")
    e_total = zij_hbm.shape[0]
    epw = e_total // _NW
    nchunks = epw // chunk

    pltpu.sync_copy(params_hbm, params_v)
    pltpu.sync_copy(centres_hbm, centres_v)

    base_w = wid * epw

    def do_chunk(c, carry):
        base = base_w + c * chunk
        pltpu.sync_copy(zij_hbm.at[pl.ds(base, chunk)], zij_v)
        pltpu.sync_copy(rij_hbm.at[pl.ds(base, chunk), :], rij_v)

        def do_group(g, gcarry):
            lbl_vec = zij_v[pl.ds(g * _LANES, _LANES)]
            for k in range(_LANES):
                e = g * _LANES + k
                lbl = lbl_vec[k]
                rs = jnp.sum(rij_v[e, :])
                pv = params_v[lbl, :]
                ww = pv[0]
                sgm = pv[1]
                for j in range(n_out_vecs):
                    cc = centres_v[lbl, pl.ds(j * _LANES, _LANES)]
                    t = (rs - cc) * sgm
                    out_v[e, pl.ds(j * _LANES, _LANES)] = ww * jnp.exp(-(t * t))
            return gcarry

        lax.fori_loop(0, chunk // _LANES, do_group, 0)
        pltpu.sync_copy(out_v, out_hbm.at[pl.ds(base, chunk), :])
        return carry

    lax.fori_loop(0, nchunks, do_chunk, 0)


@jax.jit
def kernel(zij_label, rij, nuww, sigmas, centres):
    e_total, c_in = rij.shape
    n_labels, out_f = centres.shape
    assert e_total % _NW == 0
    epw = e_total // _NW
    chunk = 400
    if epw % chunk != 0:
        chunk = epw
    assert out_f % _LANES == 0 and c_in == _LANES

    params = jnp.zeros((n_labels, _LANES), jnp.float32)
    params = params.at[:, 0].set(nuww).at[:, 1].set(sigmas)

    mesh = plsc.VectorSubcoreMesh(
        core_axis_name="c", subcore_axis_name="s",
        num_cores=_NC, num_subcores=_NS)
    body = functools.partial(_rconv_body, chunk, out_f // _LANES)
    run = pl.kernel(
        body,
        out_type=jax.ShapeDtypeStruct((e_total, out_f), jnp.float32),
        mesh=mesh,
        compiler_params=pltpu.CompilerParams(needs_layout_passes=False),
        scratch_types=[
            pltpu.VMEM((n_labels, _LANES), jnp.float32),
            pltpu.VMEM((n_labels, out_f), jnp.float32),
            pltpu.VMEM((chunk,), jnp.int32),
            pltpu.VMEM((chunk, c_in), jnp.float32),
            pltpu.VMEM((chunk, out_f), jnp.float32),
        ],
    )
    return run(zij_label.astype(jnp.int32), rij, params, centres)


# speedup vs baseline: 8.2553x; 2.8711x over previous
import functools

import jax
import jax.numpy as jnp
from jax import lax
from jax.experimental import pallas as pl
from jax.experimental.pallas import tpu as pltpu
from jax.experimental.pallas import tpu_sc as plsc

_NC = 2
_NS = 16
_NW = _NC * _NS
_LANES = 16


def _rconv_body(chunk, n_out_vecs,
                zij_hbm, rij_hbm, params_hbm, ctab_hbm,
                out_hbm,
                params_v, ctab_v, zij_v, rij_v, out_v):
    wid = lax.axis_index("s") * _NC + lax.axis_index("c")
    e_total = zij_hbm.shape[0]
    epw = e_total // _NW
    nchunks = epw // chunk

    pltpu.sync_copy(params_hbm, params_v)
    pltpu.sync_copy(ctab_hbm, ctab_v)

    base_w = wid * epw

    def do_chunk(c, carry):
        base = base_w + c * chunk
        pltpu.sync_copy(zij_hbm.at[pl.ds(base, chunk)], zij_v)
        pltpu.sync_copy(rij_hbm.at[pl.ds(base, chunk), :], rij_v)

        lanes_iota = lax.iota(jnp.int32, _LANES)

        @plsc.parallel_loop(0, chunk // _LANES)
        def do_group(g):
            lbl_vec = zij_v[pl.ds(g * _LANES, _LANES)]
            wwv = plsc.load_gather(
                params_v, [lbl_vec, jnp.zeros((_LANES,), jnp.int32)])
            sgv = plsc.load_gather(
                params_v, [lbl_vec, jnp.ones((_LANES,), jnp.int32)])
            erow = g * _LANES + lanes_iota
            rsv = plsc.load_gather(
                rij_v, [erow, jnp.zeros((_LANES,), jnp.int32)])
            for cch in range(1, _LANES):
                rsv = rsv + plsc.load_gather(
                    rij_v, [erow, jnp.full((_LANES,), cch, jnp.int32)])
            bqv = rsv * sgv
            for k in range(_LANES):
                e = g * _LANES + k
                lbl = lbl_vec[k]
                ww = wwv[k]
                bq = bqv[k]
                avs = [ctab_v[lbl, pl.ds(j * _LANES, _LANES)]
                       for j in range(n_out_vecs)]
                us = [(bq - a) * (a - bq) for a in avs]
                evs = [jnp.exp(u) for u in us]
                for j in range(n_out_vecs):
                    out_v[e, pl.ds(j * _LANES, _LANES)] = ww * evs[j]

        del do_group
        pltpu.sync_copy(out_v, out_hbm.at[pl.ds(base, chunk), :])
        return carry

    lax.fori_loop(0, nchunks, do_chunk, 0)


@jax.jit
def kernel(zij_label, rij, nuww, sigmas, centres):
    e_total, c_in = rij.shape
    n_labels, out_f = centres.shape
    assert e_total % _NW == 0
    epw = e_total // _NW
    chunk = 400
    if epw % chunk != 0:
        chunk = epw
    assert out_f % _LANES == 0 and c_in == _LANES

    params = jnp.zeros((n_labels, _LANES), jnp.float32)
    params = params.at[:, 0].set(nuww).at[:, 1].set(sigmas)
    ctab = centres * sigmas[:, None]

    mesh = plsc.VectorSubcoreMesh(
        core_axis_name="c", subcore_axis_name="s",
        num_cores=_NC, num_subcores=_NS)
    body = functools.partial(_rconv_body, chunk, out_f // _LANES)
    run = pl.kernel(
        body,
        out_type=jax.ShapeDtypeStruct((e_total, out_f), jnp.float32),
        mesh=mesh,
        compiler_params=pltpu.CompilerParams(needs_layout_passes=False),
        scratch_types=[
            pltpu.VMEM((n_labels, _LANES), jnp.float32),
            pltpu.VMEM((n_labels, out_f), jnp.float32),
            pltpu.VMEM((chunk,), jnp.int32),
            pltpu.VMEM((chunk, c_in), jnp.float32),
            pltpu.VMEM((chunk, out_f), jnp.float32),
        ],
    )
    return run(zij_label.astype(jnp.int32), rij, params, ctab)


# speedup vs baseline: 10.8740x; 1.3172x over previous
import functools

import jax
import jax.numpy as jnp
from jax import lax
from jax.experimental import pallas as pl
from jax.experimental.pallas import tpu as pltpu
from jax.experimental.pallas import tpu_sc as plsc

_NC = 2
_NS = 16
_NW = _NC * _NS
_LANES = 16


def _rconv_body(chunk, n_out_vecs,
                zij_hbm, rij_hbm, params_hbm, ctab_hbm,
                out_hbm,
                params_v, ctab_v, zij_v0, zij_v1, rij_v0, rij_v1,
                out_v0, out_v1,
                in_sem0, in_sem1, out_sem0, out_sem1):
    wid = lax.axis_index("s") * _NC + lax.axis_index("c")
    e_total = zij_hbm.shape[0]
    epw = e_total // _NW
    nchunks = epw // chunk
    in_sems = (in_sem0, in_sem1)
    out_sems = (out_sem0, out_sem1)
    zij_bufs = (zij_v0, zij_v1)
    rij_bufs = (rij_v0, rij_v1)
    out_bufs = (out_v0, out_v1)

    pltpu.sync_copy(params_hbm, params_v)
    pltpu.sync_copy(ctab_hbm, ctab_v)

    base_w = wid * epw
    lanes_iota = lax.iota(jnp.int32, _LANES)

    def start_in(c, p):
        base = base_w + c * chunk
        pltpu.async_copy(zij_hbm.at[pl.ds(base, chunk)], zij_bufs[p],
                         in_sems[p])
        pltpu.async_copy(rij_hbm.at[pl.ds(base, chunk), :], rij_bufs[p],
                         in_sems[p])

    def wait_in(c, p):
        base = base_w + c * chunk
        pltpu.make_async_copy(zij_hbm.at[pl.ds(base, chunk)], zij_bufs[p],
                              in_sems[p]).wait()
        pltpu.make_async_copy(rij_hbm.at[pl.ds(base, chunk), :], rij_bufs[p],
                              in_sems[p]).wait()

    def start_out(c, p):
        base = base_w + c * chunk
        pltpu.async_copy(out_bufs[p], out_hbm.at[pl.ds(base, chunk), :],
                         out_sems[p])

    def wait_out(c, p):
        base = base_w + c * chunk
        pltpu.make_async_copy(out_bufs[p], out_hbm.at[pl.ds(base, chunk), :],
                              out_sems[p]).wait()

    def compute_chunk(p):
        zij_p = zij_bufs[p]
        rij_p = rij_bufs[p]
        out_p = out_bufs[p]

        @plsc.parallel_loop(0, chunk // _LANES)
        def do_group(g):
            lbl_vec = zij_p[pl.ds(g * _LANES, _LANES)]
            wwv = plsc.load_gather(
                params_v, [lbl_vec, jnp.zeros((_LANES,), jnp.int32)])
            sgv = plsc.load_gather(
                params_v, [lbl_vec, jnp.ones((_LANES,), jnp.int32)])
            erow = g * _LANES + lanes_iota
            rsv = plsc.load_gather(
                rij_p, [erow, jnp.zeros((_LANES,), jnp.int32)])
            for cch in range(1, _LANES):
                rsv = rsv + plsc.load_gather(
                    rij_p, [erow, jnp.full((_LANES,), cch, jnp.int32)])
            bqv = rsv * sgv
            for k in range(_LANES):
                e = g * _LANES + k
                lbl = lbl_vec[k]
                ww = wwv[k]
                bq = bqv[k]
                avs = [ctab_v[lbl, pl.ds(j * _LANES, _LANES)]
                       for j in range(n_out_vecs)]
                us = [(bq - a) * (a - bq) for a in avs]
                evs = [jnp.exp(u) for u in us]
                for j in range(n_out_vecs):
                    out_p[e, pl.ds(j * _LANES, _LANES)] = ww * evs[j]

        del do_group

    npairs = nchunks // 2
    start_in(0, 0)
    if nchunks > 1:
        start_in(1, 1)

    def do_pair(i, carry):
        for p in range(2):
            c = 2 * i + p
            wait_in(c, p)

            @pl.when(i >= 1)
            def _():
                wait_out(c - 2, p)

            compute_chunk(p)
            start_out(c, p)

            @pl.when(2 * i + p + 2 < nchunks)
            def _():
                start_in(c + 2, p)

        return carry

    lax.fori_loop(0, npairs, do_pair, 0)

    if nchunks % 2 == 1:
        c = nchunks - 1
        wait_in(c, 0)
        if nchunks > 2:
            wait_out(c - 2, 0)
        compute_chunk(0)
        start_out(c, 0)
        if nchunks > 1:
            wait_out(c - 1, 1)
        wait_out(c, 0)
    else:
        if nchunks > 1:
            wait_out(nchunks - 2, 0)
        wait_out(nchunks - 1, 1 if nchunks > 1 else 0)


@jax.jit
def kernel(zij_label, rij, nuww, sigmas, centres):
    e_total, c_in = rij.shape
    n_labels, out_f = centres.shape
    assert e_total % _NW == 0
    epw = e_total // _NW
    chunk = 80
    if epw % chunk != 0:
        chunk = epw
    assert out_f % _LANES == 0 and c_in == _LANES

    params = jnp.zeros((n_labels, _LANES), jnp.float32)
    params = params.at[:, 0].set(nuww).at[:, 1].set(sigmas)
    ctab = centres * sigmas[:, None]

    mesh = plsc.VectorSubcoreMesh(
        core_axis_name="c", subcore_axis_name="s",
        num_cores=_NC, num_subcores=_NS)
    body = functools.partial(_rconv_body, chunk, out_f // _LANES)
    run = pl.kernel(
        body,
        out_type=jax.ShapeDtypeStruct((e_total, out_f), jnp.float32),
        mesh=mesh,
        compiler_params=pltpu.CompilerParams(needs_layout_passes=False),
        scratch_types=[
            pltpu.VMEM((n_labels, _LANES), jnp.float32),
            pltpu.VMEM((n_labels, out_f), jnp.float32),
            pltpu.VMEM((chunk,), jnp.int32),
            pltpu.VMEM((chunk,), jnp.int32),
            pltpu.VMEM((chunk, c_in), jnp.float32),
            pltpu.VMEM((chunk, c_in), jnp.float32),
            pltpu.VMEM((chunk, out_f), jnp.float32),
            pltpu.VMEM((chunk, out_f), jnp.float32),
            pltpu.SemaphoreType.DMA,
            pltpu.SemaphoreType.DMA,
            pltpu.SemaphoreType.DMA,
            pltpu.SemaphoreType.DMA,
        ],
    )
    return run(zij_label.astype(jnp.int32), rij, params, ctab)
